# baseline (speedup 1.0000x reference)
import functools

import jax
import jax.numpy as jnp
from jax import lax
from jax.experimental import pallas as pl
from jax.experimental.pallas import tpu as pltpu
from jax.experimental.pallas import tpu_sc as plsc

NC, NS = 2, 16
NW = NC * NS
CHUNK = 128


def _repack_body(u_ref, v_ref, o_ref):
    E = u_ref.shape[0]
    bc = u_ref.shape[1]
    eye = (lax.broadcasted_iota(jnp.int32, (E, E), 0)
           == lax.broadcasted_iota(jnp.int32, (E, E), 1)).astype(jnp.float32)
    dn = (((0,), (0,)), ((), ()))
    ut = lax.dot_general(u_ref[...], eye, dn, preferred_element_type=jnp.float32)
    vt = lax.dot_general(v_ref[...], eye, dn, preferred_element_type=jnp.float32)
    fused = jnp.concatenate([ut, vt], axis=1)

    def rne_bf16_bits(x):
        b = lax.bitcast_convert_type(x, jnp.uint32)
        return (b + jnp.uint32(0x7FFF) + ((b >> 16) & jnp.uint32(1))) >> 16

    lo = rne_bf16_bits(fused[: bc // 2])
    hi = rne_bf16_bits(fused[bc // 2:])
    o_ref[...] = lax.bitcast_convert_type(lo | (hi << 16), jnp.float32)


def _repack(uT, vT, bc):
    E, N = uT.shape
    grid = (N + bc - 1) // bc
    return pl.pallas_call(
        _repack_body,
        grid=(grid,),
        in_specs=[
            pl.BlockSpec((E, bc), lambda i: (0, i)),
            pl.BlockSpec((E, bc), lambda i: (0, i)),
        ],
        out_specs=pl.BlockSpec((bc // 2, 2 * E), lambda i: (i, 0)),
        out_shape=jax.ShapeDtypeStruct((grid * (bc // 2), 2 * E), jnp.float32),
        compiler_params=pltpu.CompilerParams(vmem_limit_bytes=60 * 1024 * 1024),
    )(uT, vT)


def _make_sc_gather(B, E2):
    bpw = B // NW
    kch = bpw // CHUNK
    mesh = plsc.VectorSubcoreMesh(
        core_axis_name="c", subcore_axis_name="s", num_cores=NC, num_subcores=NS
    )

    @functools.partial(
        pl.kernel,
        out_type=(
            jax.ShapeDtypeStruct((B, E2), jnp.float32),
            jax.ShapeDtypeStruct((B, E2), jnp.float32),
        ),
        mesh=mesh,
        compiler_params=pltpu.CompilerParams(use_tc_tiling_on_sc=False),
        scratch_types=[
            pltpu.VMEM((kch, CHUNK), jnp.int32),
            pltpu.VMEM((kch, CHUNK), jnp.int32),
            pltpu.VMEM((bpw, E2), jnp.float32),
            pltpu.SemaphoreType.DMA,
        ],
    )
    def sc_gather(user_hbm, item_hbm, rt_hbm, u_out, v_out,
                  uidx_v, iidx_v, rows_v, sem):
        wid = lax.axis_index("s") * NC + lax.axis_index("c")
        base = wid * bpw
        rb = wid * kch
        pltpu.sync_copy(user_hbm.at[pl.ds(rb, kch)], uidx_v)
        pltpu.sync_copy(item_hbm.at[pl.ds(rb, kch)], iidx_v)
        cps = []
        for j in range(kch):
            cps.append(pltpu.async_copy(
                rt_hbm.at[uidx_v.at[j]],
                rows_v.at[pl.ds(j * CHUNK, CHUNK)], sem))
        for c in cps:
            c.wait()
        pltpu.sync_copy(rows_v, u_out.at[pl.ds(base, bpw)])
        cps = []
        for j in range(kch):
            cps.append(pltpu.async_copy(
                rt_hbm.at[iidx_v.at[j]],
                rows_v.at[pl.ds(j * CHUNK, CHUNK)], sem))
        for c in cps:
            c.wait()
        pltpu.sync_copy(rows_v, v_out.at[pl.ds(base, bpw)])

    return sc_gather


def _unpack_parity(g_ref, p_ref):
    w = lax.bitcast_convert_type(g_ref[...], jnp.uint32)
    even = lax.bitcast_convert_type(w << 16, jnp.float32)
    odd = lax.bitcast_convert_type(w & jnp.uint32(0xFFFF0000), jnp.float32)
    return jnp.where(p_ref[...] > 0, odd, even)


def _mlp_body(gu_ref, gv_ref, pu_ref, pv_ref, w1u_ref, w1v_ref, b1_ref,
              w2_ref, b2_ref, w3_ref, b3_ref, w4_ref, b4_ref, o_ref):
    E = w1u_ref.shape[0]
    xu = _unpack_parity(gu_ref, pu_ref)
    xv = _unpack_parity(gv_ref, pv_ref)
    u = xu[:, :E]
    v = xv[:, E:]
    x = jnp.dot(u, w1u_ref[...], preferred_element_type=jnp.float32)
    x = x + jnp.dot(v, w1v_ref[...], preferred_element_type=jnp.float32)
    x = jnp.maximum(x + b1_ref[...], 0.0)
    x = jnp.dot(x, w2_ref[...], preferred_element_type=jnp.float32)
    x = jnp.maximum(x + b2_ref[...], 0.0)
    x = jnp.dot(x, w3_ref[...], preferred_element_type=jnp.float32)
    x = jnp.maximum(x + b3_ref[...], 0.0)
    o_ref[...] = jnp.sum(x * w4_ref[...], axis=1, keepdims=True) + b4_ref[...]


def _mlp(gu, gv, pu, pv, w1u, w1v, b1, w2, b2, w3, b3, w4, b4, bblk):
    B, E2 = gu.shape
    grid = B // bblk
    full = lambda shape: pl.BlockSpec(shape, lambda i: (0, 0))
    return pl.pallas_call(
        _mlp_body,
        grid=(grid,),
        in_specs=[
            pl.BlockSpec((bblk, E2), lambda i: (i, 0)),
            pl.BlockSpec((bblk, E2), lambda i: (i, 0)),
            pl.BlockSpec((bblk, 1), lambda i: (i, 0)),
            pl.BlockSpec((bblk, 1), lambda i: (i, 0)),
            full(w1u.shape), full(w1v.shape), full(b1.shape),
            full(w2.shape), full(b2.shape),
            full(w3.shape), full(b3.shape),
            full(w4.shape), full(b4.shape),
        ],
        out_specs=pl.BlockSpec((bblk, 1), lambda i: (i, 0)),
        out_shape=jax.ShapeDtypeStruct((B, 1), jnp.float32),
    )(gu, gv, pu, pv, w1u, w1v, b1, w2, b2, w3, b3, w4, b4)


def kernel(user, item, user_emb, item_emb, W1, b1, W2, b2, W3, b3, W4, b4):
    B = user.shape[0]
    E = user_emb.shape[1]
    rt = _repack(user_emb.T, item_emb.T, bc=16384)
    useri = user.astype(jnp.int32)
    itemi = item.astype(jnp.int32)
    ku = (useri >> 14) * 8192 + (useri & 8191)
    ki = (itemi >> 14) * 8192 + (itemi & 8191)
    user2 = ku.reshape(B // CHUNK, CHUNK)
    item2 = ki.reshape(B // CHUNK, CHUNK)
    pu = ((useri >> 13) & 1).reshape(B, 1)
    pv = ((itemi >> 13) & 1).reshape(B, 1)
    gu, gv = _make_sc_gather(B, 2 * E)(user2, item2, rt)
    out = _mlp(
        gu, gv, pu, pv,
        W1[:, :E].T, W1[:, E:].T, b1.reshape(1, -1),
        W2.T, b2.reshape(1, -1),
        W3.T, b3.reshape(1, -1),
        W4.reshape(1, -1), b4.reshape(1, 1),
        bblk=2048,
    )
    return out.reshape(B)


# speedup vs baseline: 1.0170x; 1.0170x over previous
import functools

import jax
import jax.numpy as jnp
from jax import lax
from jax.experimental import pallas as pl
from jax.experimental.pallas import tpu as pltpu
from jax.experimental.pallas import tpu_sc as plsc

NC, NS = 2, 16
NW = NC * NS
CHUNK = 128


def _repack_body(u_ref, v_ref, o_ref):
    E = u_ref.shape[0]
    bc = u_ref.shape[1]
    eye = (lax.broadcasted_iota(jnp.int32, (E, E), 0)
           == lax.broadcasted_iota(jnp.int32, (E, E), 1)).astype(jnp.float32)
    dn = (((0,), (0,)), ((), ()))
    ut = lax.dot_general(u_ref[...], eye, dn, preferred_element_type=jnp.float32)
    vt = lax.dot_general(v_ref[...], eye, dn, preferred_element_type=jnp.float32)
    fused = jnp.concatenate([ut, vt], axis=1)

    lo = lax.bitcast_convert_type(fused[: bc // 2], jnp.uint32) >> 16
    hi = lax.bitcast_convert_type(fused[bc // 2:], jnp.uint32)
    o_ref[...] = lax.bitcast_convert_type(
        lo | (hi & jnp.uint32(0xFFFF0000)), jnp.float32)


def _repack(uT, vT, bc):
    E, N = uT.shape
    grid = (N + bc - 1) // bc
    return pl.pallas_call(
        _repack_body,
        grid=(grid,),
        in_specs=[
            pl.BlockSpec((E, bc), lambda i: (0, i)),
            pl.BlockSpec((E, bc), lambda i: (0, i)),
        ],
        out_specs=pl.BlockSpec((bc // 2, 2 * E), lambda i: (i, 0)),
        out_shape=jax.ShapeDtypeStruct((grid * (bc // 2), 2 * E), jnp.float32),
        compiler_params=pltpu.CompilerParams(vmem_limit_bytes=60 * 1024 * 1024),
    )(uT, vT)


def _make_sc_gather(B, E2):
    bpw = B // NW
    kch = bpw // CHUNK
    mesh = plsc.VectorSubcoreMesh(
        core_axis_name="c", subcore_axis_name="s", num_cores=NC, num_subcores=NS
    )

    @functools.partial(
        pl.kernel,
        out_type=(
            jax.ShapeDtypeStruct((B, E2), jnp.float32),
            jax.ShapeDtypeStruct((B, E2), jnp.float32),
        ),
        mesh=mesh,
        compiler_params=pltpu.CompilerParams(use_tc_tiling_on_sc=False),
        scratch_types=[
            pltpu.VMEM((kch, CHUNK), jnp.int32),
            pltpu.VMEM((kch, CHUNK), jnp.int32),
            pltpu.VMEM((bpw, E2), jnp.float32),
            pltpu.SemaphoreType.DMA,
        ],
    )
    def sc_gather(user_hbm, item_hbm, rt_hbm, u_out, v_out,
                  uidx_v, iidx_v, rows_v, sem):
        wid = lax.axis_index("s") * NC + lax.axis_index("c")
        base = wid * bpw
        rb = wid * kch
        pltpu.sync_copy(user_hbm.at[pl.ds(rb, kch)], uidx_v)
        pltpu.sync_copy(item_hbm.at[pl.ds(rb, kch)], iidx_v)
        cps = []
        for j in range(kch):
            cps.append(pltpu.async_copy(
                rt_hbm.at[uidx_v.at[j]],
                rows_v.at[pl.ds(j * CHUNK, CHUNK)], sem))
        for c in cps:
            c.wait()
        pltpu.sync_copy(rows_v, u_out.at[pl.ds(base, bpw)])
        cps = []
        for j in range(kch):
            cps.append(pltpu.async_copy(
                rt_hbm.at[iidx_v.at[j]],
                rows_v.at[pl.ds(j * CHUNK, CHUNK)], sem))
        for c in cps:
            c.wait()
        pltpu.sync_copy(rows_v, v_out.at[pl.ds(base, bpw)])

    return sc_gather


def _unpack_parity(g_ref, p_ref):
    w = lax.bitcast_convert_type(g_ref[...], jnp.uint32)
    even = lax.bitcast_convert_type(w << 16, jnp.float32)
    odd = lax.bitcast_convert_type(w & jnp.uint32(0xFFFF0000), jnp.float32)
    return jnp.where(p_ref[...] > 0, odd, even)


def _mlp_body(gu_ref, gv_ref, pu_ref, pv_ref, w1u_ref, w1v_ref, b1_ref,
              w2_ref, b2_ref, w3_ref, b3_ref, w4_ref, b4_ref, o_ref):
    E = w1u_ref.shape[0]
    xu = _unpack_parity(gu_ref, pu_ref)
    xv = _unpack_parity(gv_ref, pv_ref)
    u = xu[:, :E]
    v = xv[:, E:]
    x = jnp.dot(u, w1u_ref[...], preferred_element_type=jnp.float32)
    x = x + jnp.dot(v, w1v_ref[...], preferred_element_type=jnp.float32)
    x = jnp.maximum(x + b1_ref[...], 0.0)
    x = jnp.dot(x, w2_ref[...], preferred_element_type=jnp.float32)
    x = jnp.maximum(x + b2_ref[...], 0.0)
    x = jnp.dot(x, w3_ref[...], preferred_element_type=jnp.float32)
    x = jnp.maximum(x + b3_ref[...], 0.0)
    o_ref[...] = jnp.sum(x * w4_ref[...], axis=1, keepdims=True) + b4_ref[...]


def _mlp(gu, gv, pu, pv, w1u, w1v, b1, w2, b2, w3, b3, w4, b4, bblk):
    B, E2 = gu.shape
    grid = B // bblk
    full = lambda shape: pl.BlockSpec(shape, lambda i: (0, 0))
    return pl.pallas_call(
        _mlp_body,
        grid=(grid,),
        in_specs=[
            pl.BlockSpec((bblk, E2), lambda i: (i, 0)),
            pl.BlockSpec((bblk, E2), lambda i: (i, 0)),
            pl.BlockSpec((bblk, 1), lambda i: (i, 0)),
            pl.BlockSpec((bblk, 1), lambda i: (i, 0)),
            full(w1u.shape), full(w1v.shape), full(b1.shape),
            full(w2.shape), full(b2.shape),
            full(w3.shape), full(b3.shape),
            full(w4.shape), full(b4.shape),
        ],
        out_specs=pl.BlockSpec((bblk, 1), lambda i: (i, 0)),
        out_shape=jax.ShapeDtypeStruct((B, 1), jnp.float32),
    )(gu, gv, pu, pv, w1u, w1v, b1, w2, b2, w3, b3, w4, b4)


def kernel(user, item, user_emb, item_emb, W1, b1, W2, b2, W3, b3, W4, b4):
    B = user.shape[0]
    E = user_emb.shape[1]
    rt = _repack(user_emb.T, item_emb.T, bc=16384)
    useri = user.astype(jnp.int32)
    itemi = item.astype(jnp.int32)
    ku = (useri >> 14) * 8192 + (useri & 8191)
    ki = (itemi >> 14) * 8192 + (itemi & 8191)
    user2 = ku.reshape(B // CHUNK, CHUNK)
    item2 = ki.reshape(B // CHUNK, CHUNK)
    pu = ((useri >> 13) & 1).reshape(B, 1)
    pv = ((itemi >> 13) & 1).reshape(B, 1)
    gu, gv = _make_sc_gather(B, 2 * E)(user2, item2, rt)
    out = _mlp(
        gu, gv, pu, pv,
        W1[:, :E].T, W1[:, E:].T, b1.reshape(1, -1),
        W2.T, b2.reshape(1, -1),
        W3.T, b3.reshape(1, -1),
        W4.reshape(1, -1), b4.reshape(1, 1),
        bblk=2048,
    )
    return out.reshape(B)


# speedup vs baseline: 1.0633x; 1.0455x over previous
import functools

import jax
import jax.numpy as jnp
from jax import lax
from jax.experimental import pallas as pl
from jax.experimental.pallas import tpu as pltpu
from jax.experimental.pallas import tpu_sc as plsc

NC, NS = 2, 16
NW = NC * NS
CHUNK = 128


def _repack_body(u_ref, v_ref, o_ref):
    E = u_ref.shape[0]
    eye = (lax.broadcasted_iota(jnp.int32, (E, E), 0)
           == lax.broadcasted_iota(jnp.int32, (E, E), 1)).astype(jnp.float32)
    dn = (((0,), (0,)), ((), ()))
    ut = lax.dot_general(u_ref[...], eye, dn, preferred_element_type=jnp.float32)
    vt = lax.dot_general(v_ref[...], eye, dn, preferred_element_type=jnp.float32)
    o_ref[...] = jnp.concatenate([ut, vt], axis=1)


def _repack(uT, vT, bc):
    E, N = uT.shape
    grid = (N + bc - 1) // bc
    return pl.pallas_call(
        _repack_body,
        grid=(grid,),
        in_specs=[
            pl.BlockSpec((E, bc), lambda i: (0, i)),
            pl.BlockSpec((E, bc), lambda i: (0, i)),
        ],
        out_specs=pl.BlockSpec((bc, 2 * E), lambda i: (i, 0)),
        out_shape=jax.ShapeDtypeStruct((N, 2 * E), jnp.float32),
        compiler_params=pltpu.CompilerParams(vmem_limit_bytes=60*1024*1024),
    )(uT, vT)


def _make_sc_gather(B, E2):
    bpw = B // NW
    kch = bpw // CHUNK
    mesh = plsc.VectorSubcoreMesh(
        core_axis_name="c", subcore_axis_name="s", num_cores=NC, num_subcores=NS
    )

    @functools.partial(
        pl.kernel,
        out_type=(
            jax.ShapeDtypeStruct((B, E2), jnp.float32),
            jax.ShapeDtypeStruct((B, E2), jnp.float32),
        ),
        mesh=mesh,
        compiler_params=pltpu.CompilerParams(use_tc_tiling_on_sc=False),
        scratch_types=[
            pltpu.VMEM((kch, CHUNK), jnp.int32),
            pltpu.VMEM((kch, CHUNK), jnp.int32),
            pltpu.VMEM((bpw, E2), jnp.float32),
            pltpu.SemaphoreType.DMA,
        ],
    )
    def sc_gather(user_hbm, item_hbm, rt_hbm, u_out, v_out,
                  uidx_v, iidx_v, rows_v, sem):
        wid = lax.axis_index("s") * NC + lax.axis_index("c")
        base = wid * bpw
        rb = wid * kch
        pltpu.sync_copy(user_hbm.at[pl.ds(rb, kch)], uidx_v)
        pltpu.sync_copy(item_hbm.at[pl.ds(rb, kch)], iidx_v)
        cps = []
        for j in range(kch):
            cps.append(pltpu.async_copy(
                rt_hbm.at[uidx_v.at[j]],
                rows_v.at[pl.ds(j * CHUNK, CHUNK)], sem))
        for c in cps:
            c.wait()
        pltpu.sync_copy(rows_v, u_out.at[pl.ds(base, bpw)])
        cps = []
        for j in range(kch):
            cps.append(pltpu.async_copy(
                rt_hbm.at[iidx_v.at[j]],
                rows_v.at[pl.ds(j * CHUNK, CHUNK)], sem))
        for c in cps:
            c.wait()
        pltpu.sync_copy(rows_v, v_out.at[pl.ds(base, bpw)])

    return sc_gather


def _mlp_body(gu_ref, gv_ref, w1u_ref, w1v_ref, b1_ref, w2_ref, b2_ref,
              w3_ref, b3_ref, w4_ref, b4_ref, o_ref):
    E = w1u_ref.shape[0]
    u = gu_ref[:, :E]
    v = gv_ref[:, E:]
    x = jnp.dot(u, w1u_ref[...], preferred_element_type=jnp.float32)
    x = x + jnp.dot(v, w1v_ref[...], preferred_element_type=jnp.float32)
    x = jnp.maximum(x + b1_ref[...], 0.0)
    x = jnp.dot(x, w2_ref[...], preferred_element_type=jnp.float32)
    x = jnp.maximum(x + b2_ref[...], 0.0)
    x = jnp.dot(x, w3_ref[...], preferred_element_type=jnp.float32)
    x = jnp.maximum(x + b3_ref[...], 0.0)
    o_ref[...] = jnp.sum(x * w4_ref[...], axis=1, keepdims=True) + b4_ref[...]


def _mlp(gu, gv, w1u, w1v, b1, w2, b2, w3, b3, w4, b4, bblk):
    B, E2 = gu.shape
    grid = B // bblk
    full = lambda shape: pl.BlockSpec(shape, lambda i: (0, 0))
    return pl.pallas_call(
        _mlp_body,
        grid=(grid,),
        in_specs=[
            pl.BlockSpec((bblk, E2), lambda i: (i, 0)),
            pl.BlockSpec((bblk, E2), lambda i: (i, 0)),
            full(w1u.shape), full(w1v.shape), full(b1.shape),
            full(w2.shape), full(b2.shape),
            full(w3.shape), full(b3.shape),
            full(w4.shape), full(b4.shape),
        ],
        out_specs=pl.BlockSpec((bblk, 1), lambda i: (i, 0)),
        out_shape=jax.ShapeDtypeStruct((B, 1), jnp.float32),
    )(gu, gv, w1u, w1v, b1, w2, b2, w3, b3, w4, b4)


def kernel(user, item, user_emb, item_emb, W1, b1, W2, b2, W3, b3, W4, b4):
    B = user.shape[0]
    E = user_emb.shape[1]
    rt = _repack(user_emb.T, item_emb.T, bc=20480)
    user2 = user.astype(jnp.int32).reshape(B // CHUNK, CHUNK)
    item2 = item.astype(jnp.int32).reshape(B // CHUNK, CHUNK)
    gu, gv = _make_sc_gather(B, 2 * E)(user2, item2, rt)
    out = _mlp(
        gu, gv,
        W1[:, :E].T, W1[:, E:].T, b1.reshape(1, -1),
        W2.T, b2.reshape(1, -1),
        W3.T, b3.reshape(1, -1),
        W4.reshape(1, -1), b4.reshape(1, 1),
        bblk=4096,
    )
    return out.reshape(B)
